# Initial kernel scaffold; baseline (speedup 1.0000x reference)
import jax
import jax.numpy as jnp
from jax.experimental import pallas as pl


def kernel(k_cache, v_cache, pos_ids, k, v):
    raise NotImplementedError("write your pallas kernel here")


# speedup vs baseline: 1.8167x; 1.8167x over previous
import jax
import jax.numpy as jnp
from jax.experimental import pallas as pl
from jax.experimental.pallas import tpu as pltpu

_N_KV_HEADS = 8
_MAX_CONTEXT = 8192
_HEAD_DIM = 128
_S = 16
_RB = 1024


def _body(pos_ref, kc_ref, k_ref, vc_ref, v_ref, ko_ref, vo_ref):
    j = pl.program_id(1)
    base = j * _RB
    ko_ref[...] = kc_ref[...]
    vo_ref[...] = vc_ref[...]
    for s in range(_S):
        local = pos_ref[s] - base

        @pl.when((local >= 0) & (local < _RB))
        def _():
            ko_ref[0, pl.ds(local, 1), :] = k_ref[0, pl.ds(s, 1), :]
            vo_ref[0, pl.ds(local, 1), :] = v_ref[0, pl.ds(s, 1), :]


def kernel(k_cache, v_cache, pos_ids, k, v):
    kc = k_cache[0]
    vc = v_cache[0]
    kk = k[0]
    vv = v[0]
    pos = pos_ids.astype(jnp.int32)

    grid = (_N_KV_HEADS, _MAX_CONTEXT // _RB)
    cache_spec = pl.BlockSpec((1, _RB, _HEAD_DIM), lambda h, j, pos_ref: (h, j, 0))
    new_spec = pl.BlockSpec((1, _S, _HEAD_DIM), lambda h, j, pos_ref: (h, 0, 0))

    ko, vo = pl.pallas_call(
        _body,
        grid_spec=pltpu.PrefetchScalarGridSpec(
            num_scalar_prefetch=1,
            grid=grid,
            in_specs=[cache_spec, new_spec, cache_spec, new_spec],
            out_specs=[cache_spec, cache_spec],
        ),
        out_shape=[
            jax.ShapeDtypeStruct(kc.shape, kc.dtype),
            jax.ShapeDtypeStruct(vc.shape, vc.dtype),
        ],
    )(pos, kc, kk, vc, vv)
    return (ko[None], vo[None])


# speedup vs baseline: 2.6035x; 1.4331x over previous
import jax
import jax.numpy as jnp
from jax.experimental import pallas as pl
from jax.experimental.pallas import tpu as pltpu

_N_KV_HEADS = 8
_MAX_CONTEXT = 8192
_HEAD_DIM = 128
_S = 16
_RB = 2048


def _body(pos_ref, kc_ref, k_ref, vc_ref, v_ref, ko_ref, vo_ref):
    j = pl.program_id(1)
    base = j * _RB
    ko_ref[...] = kc_ref[...]
    vo_ref[...] = vc_ref[...]
    for s in range(_S):
        local = pos_ref[s] - base

        @pl.when((local >= 0) & (local < _RB))
        def _():
            ko_ref[0, pl.ds(local, 1), :] = k_ref[0, pl.ds(s, 1), :]
            vo_ref[0, pl.ds(local, 1), :] = v_ref[0, pl.ds(s, 1), :]


def kernel(k_cache, v_cache, pos_ids, k, v):
    kc = k_cache[0]
    vc = v_cache[0]
    kk = k[0]
    vv = v[0]
    pos = pos_ids.astype(jnp.int32)

    grid = (_N_KV_HEADS, _MAX_CONTEXT // _RB)
    cache_spec = pl.BlockSpec((1, _RB, _HEAD_DIM), lambda h, j, pos_ref: (h, j, 0))
    new_spec = pl.BlockSpec((1, _S, _HEAD_DIM), lambda h, j, pos_ref: (h, 0, 0))

    ko, vo = pl.pallas_call(
        _body,
        grid_spec=pltpu.PrefetchScalarGridSpec(
            num_scalar_prefetch=1,
            grid=grid,
            in_specs=[cache_spec, new_spec, cache_spec, new_spec],
            out_specs=[cache_spec, cache_spec],
        ),
        out_shape=[
            jax.ShapeDtypeStruct(kc.shape, kc.dtype),
            jax.ShapeDtypeStruct(vc.shape, vc.dtype),
        ],
    )(pos, kc, kk, vc, vv)
    return (ko[None], vo[None])


# speedup vs baseline: 2.8687x; 1.1019x over previous
import jax
import jax.numpy as jnp
from jax.experimental import pallas as pl
from jax.experimental.pallas import tpu as pltpu

_N_KV_HEADS = 8
_MAX_CONTEXT = 8192
_HEAD_DIM = 128
_S = 16
_RB = 4096


def _body(pos_ref, kc_ref, k_ref, vc_ref, v_ref, ko_ref, vo_ref):
    j = pl.program_id(1)
    base = j * _RB
    ko_ref[...] = kc_ref[...]
    vo_ref[...] = vc_ref[...]
    for s in range(_S):
        local = pos_ref[s] - base

        @pl.when((local >= 0) & (local < _RB))
        def _():
            ko_ref[0, pl.ds(local, 1), :] = k_ref[0, pl.ds(s, 1), :]
            vo_ref[0, pl.ds(local, 1), :] = v_ref[0, pl.ds(s, 1), :]


def kernel(k_cache, v_cache, pos_ids, k, v):
    kc = k_cache[0]
    vc = v_cache[0]
    kk = k[0]
    vv = v[0]
    pos = pos_ids.astype(jnp.int32)

    grid = (_N_KV_HEADS, _MAX_CONTEXT // _RB)
    cache_spec = pl.BlockSpec((1, _RB, _HEAD_DIM), lambda h, j, pos_ref: (h, j, 0))
    new_spec = pl.BlockSpec((1, _S, _HEAD_DIM), lambda h, j, pos_ref: (h, 0, 0))

    ko, vo = pl.pallas_call(
        _body,
        grid_spec=pltpu.PrefetchScalarGridSpec(
            num_scalar_prefetch=1,
            grid=grid,
            in_specs=[cache_spec, new_spec, cache_spec, new_spec],
            out_specs=[cache_spec, cache_spec],
        ),
        out_shape=[
            jax.ShapeDtypeStruct(kc.shape, kc.dtype),
            jax.ShapeDtypeStruct(vc.shape, vc.dtype),
        ],
    )(pos, kc, kk, vc, vv)
    return (ko[None], vo[None])


# speedup vs baseline: 2.9684x; 1.0348x over previous
import jax
import jax.numpy as jnp
from jax.experimental import pallas as pl
from jax.experimental.pallas import tpu as pltpu

_N_KV_HEADS = 8
_MAX_CONTEXT = 8192
_HEAD_DIM = 128
_S = 16
_RB = 8192


def _body(pos_ref, kc_ref, k_ref, vc_ref, v_ref, ko_ref, vo_ref):
    j = pl.program_id(1)
    base = j * _RB
    ko_ref[...] = kc_ref[...]
    vo_ref[...] = vc_ref[...]
    for s in range(_S):
        local = pos_ref[s] - base

        @pl.when((local >= 0) & (local < _RB))
        def _():
            ko_ref[0, pl.ds(local, 1), :] = k_ref[0, pl.ds(s, 1), :]
            vo_ref[0, pl.ds(local, 1), :] = v_ref[0, pl.ds(s, 1), :]


def kernel(k_cache, v_cache, pos_ids, k, v):
    kc = k_cache[0]
    vc = v_cache[0]
    kk = k[0]
    vv = v[0]
    pos = pos_ids.astype(jnp.int32)

    grid = (_N_KV_HEADS, _MAX_CONTEXT // _RB)
    cache_spec = pl.BlockSpec((1, _RB, _HEAD_DIM), lambda h, j, pos_ref: (h, j, 0))
    new_spec = pl.BlockSpec((1, _S, _HEAD_DIM), lambda h, j, pos_ref: (h, 0, 0))

    ko, vo = pl.pallas_call(
        _body,
        grid_spec=pltpu.PrefetchScalarGridSpec(
            num_scalar_prefetch=1,
            grid=grid,
            in_specs=[cache_spec, new_spec, cache_spec, new_spec],
            out_specs=[cache_spec, cache_spec],
        ),
        out_shape=[
            jax.ShapeDtypeStruct(kc.shape, kc.dtype),
            jax.ShapeDtypeStruct(vc.shape, vc.dtype),
        ],
    )(pos, kc, kk, vc, vv)
    return (ko[None], vo[None])
